# baseline (speedup 1.0000x reference)
import functools
import math

import jax
import jax.numpy as jnp
import numpy as np
from jax.experimental import pallas as pl
from jax.experimental.pallas import tpu as pltpu

_S = 8
_U = 8
_K = 12
_NEG = 16
_ZD = 64
_CD = 256
_NCLS = _NEG + 1


@functools.lru_cache(maxsize=None)
def _indices(length: int):
    bidx = np.zeros((_K, 1, _U * _NEG), dtype=np.int32)
    seq_all = np.zeros((_K, _S, _U * _NEG, length), dtype=np.int16)
    with jax.ensure_compile_time_eval():
        rkey = jax.random.key(42)
        for k in range(1, _K + 1):
            kb = jax.random.fold_in(rkey, 2 * k)
            ks = jax.random.fold_in(rkey, 2 * k + 1)
            batch_index = jax.random.randint(kb, (_U, _NEG), 0, _U)
            seq_index = jax.random.randint(ks, (_S, _U, _NEG, length), 1, length)
            seq_index = seq_index + jnp.arange(length)
            seq_index = jnp.remainder(seq_index, length)
            bidx[k - 1, 0] = np.asarray(batch_index).reshape(-1).astype(np.int32)
            seq_all[k - 1] = np.asarray(seq_index).reshape(_S, _U * _NEG, length).astype(np.int16)
    return bidx, seq_all


def _cpc_body(cc_ref, w_ref, b_ref, zsh_ref, bidx_ref, seq_ref,
              lossp_ref, accp_ref, p_scr0, p_scr1):
    length = seq_ref.shape[-1]
    row_i = jax.lax.broadcasted_iota(jnp.int16, (length, length), 0)
    col_i = jax.lax.broadcasted_iota(jnp.int16, (length, length), 1)
    zero = jnp.bfloat16(0.0)
    cc = cc_ref[0]

    def one_step(k, p_scr):
        wc = jnp.dot(cc, w_ref[k], preferred_element_type=jnp.float32)
        wc = (wc + b_ref[k]) * (1.0 / math.sqrt(_ZD))
        z = zsh_ref[k, 0]
        for bb in range(_U):
            zb = z[bb * length:(bb + 1) * length, :]
            p_scr[bb] = jax.lax.dot_general(
                zb, wc, (((1,), (1,)), ((), ())),
                preferred_element_type=jnp.float32).astype(jnp.bfloat16)
        for u in range(_U):
            lo, hi = u * length, (u + 1) * length
            puu = p_scr[u, :, lo:hi]
            f_pos = jnp.sum(jnp.where(row_i == col_i, puu, zero), axis=0)
            f_rows = [f_pos]
            for n in range(_NEG):
                m = u * _NEG + n
                bn = bidx_ref[k, 0, m]
                blk = p_scr[bn, :, lo:hi]
                sq = seq_ref[k, 0, m, :]
                f_rows.append(
                    jnp.sum(jnp.where(row_i == sq[None, :], blk, zero), axis=0))
            f = jnp.stack(f_rows, axis=0).astype(jnp.float32)
            mx = jnp.max(f, axis=0)
            lse = mx + jnp.log(jnp.sum(jnp.exp(f - mx), axis=0))
            lossp_ref[k, 0, u, :] = lse - f[0]
            acc = (f[0] >= jnp.max(f[1:], axis=0)).astype(jnp.float32)
            accp_ref[k, 0, u, :] = acc

    def pair(i, _):
        one_step(2 * i, p_scr0)
        one_step(2 * i + 1, p_scr1)
        return _

    jax.lax.fori_loop(0, _K // 2, pair, None)


def kernel(z, c, W, b, nframes):
    del nframes
    length = z.shape[1] - _K
    bidx_np, seq_np = _indices(length)
    bidx = jnp.asarray(bidx_np)
    seq = jnp.asarray(seq_np)

    cc = c[:, :-_K, :].reshape(_S, _U * length, _CD)
    z4 = z.reshape(_S, _U, z.shape[1], _ZD)
    zsh = jnp.stack(
        [z4[:, :, k:length + k, :].reshape(_S, _U * length, _ZD)
         for k in range(1, _K + 1)], axis=0)
    b2 = b.reshape(_K, 1, _ZD)

    grid = (_S,)
    lossp, accp = pl.pallas_call(
        _cpc_body,
        grid=grid,
        in_specs=[
            pl.BlockSpec((1, _U * length, _CD), lambda s: (s, 0, 0)),
            pl.BlockSpec((_K, _CD, _ZD), lambda s: (0, 0, 0)),
            pl.BlockSpec((_K, 1, _ZD), lambda s: (0, 0, 0)),
            pl.BlockSpec((_K, 1, _U * length, _ZD), lambda s: (0, s, 0, 0)),
            pl.BlockSpec((_K, 1, _U * _NEG), lambda s: (0, 0, 0),
                         memory_space=pltpu.SMEM),
            pl.BlockSpec((_K, 1, _U * _NEG, length), lambda s: (0, s, 0, 0)),
        ],
        out_specs=[
            pl.BlockSpec((_K, 1, _U, length), lambda s: (0, s, 0, 0)),
            pl.BlockSpec((_K, 1, _U, length), lambda s: (0, s, 0, 0)),
        ],
        out_shape=[
            jax.ShapeDtypeStruct((_K, _S, _U, length), jnp.float32),
            jax.ShapeDtypeStruct((_K, _S, _U, length), jnp.float32),
        ],
        scratch_shapes=[pltpu.VMEM((_U, length, _U * length), jnp.bfloat16),
                        pltpu.VMEM((_U, length, _U * length), jnp.bfloat16)],
    )(cc, W, b2, zsh, bidx, seq)

    denom = float(_S * _U * length)
    losses = lossp.sum(axis=(1, 2, 3)) / denom
    accs = accp.sum(axis=(1, 2, 3)) / denom
    return losses.mean(), accs


# speedup vs baseline: 1.3632x; 1.3632x over previous
import functools
import math

import jax
import jax.numpy as jnp
import numpy as np
from jax.experimental import pallas as pl
from jax.experimental.pallas import tpu as pltpu

_S = 8
_U = 8
_K = 12
_NEG = 16
_ZD = 64
_CD = 256
_NCLS = _NEG + 1


@functools.lru_cache(maxsize=None)
def _indices(length: int):
    bidx = np.zeros((_K, _U * _NEG), dtype=np.int32)
    seq_all = np.zeros((_K, _S, _U * _NEG, length), dtype=np.int32)
    with jax.ensure_compile_time_eval():
        rkey = jax.random.key(42)
        for k in range(1, _K + 1):
            kb = jax.random.fold_in(rkey, 2 * k)
            ks = jax.random.fold_in(rkey, 2 * k + 1)
            batch_index = jax.random.randint(kb, (_U, _NEG), 0, _U)
            seq_index = jax.random.randint(ks, (_S, _U, _NEG, length), 1, length)
            seq_index = seq_index + jnp.arange(length)
            seq_index = jnp.remainder(seq_index, length)
            bidx[k - 1] = np.asarray(batch_index).reshape(-1).astype(np.int32)
            seq_all[k - 1] = np.asarray(seq_index).reshape(_S, _U * _NEG, length).astype(np.int32)
    return bidx, seq_all


def _make_body(bidx):
    def _cpc_body(cc_ref, w_ref, b_ref, zsh_ref, seq_ref,
                  lossp_ref, accp_ref, p_scr0, p_scr1):
        length = seq_ref.shape[-1]
        row_i = jax.lax.broadcasted_iota(jnp.int32, (length, length), 0)
        col_i = jax.lax.broadcasted_iota(jnp.int32, (length, length), 1)
        cc = cc_ref[0]

        for k in range(_K):
            p_scr = p_scr0 if k % 2 == 0 else p_scr1
            wc = jnp.dot(cc, w_ref[k], preferred_element_type=jnp.float32)
            wc = (wc + b_ref[k]) * (1.0 / math.sqrt(_ZD))
            z = zsh_ref[k, 0]
            for bb in range(_U):
                zb = z[bb * length:(bb + 1) * length, :]
                p_scr[bb] = jax.lax.dot_general(
                    zb, wc, (((1,), (1,)), ((), ())),
                    preferred_element_type=jnp.float32)
            for u in range(_U):
                lo, hi = u * length, (u + 1) * length
                needed = sorted({u} | {int(bidx[k, u * _NEG + n])
                                       for n in range(_NEG)})
                pages = {b: p_scr[b, :, lo:hi] for b in needed}
                f_pos = jnp.sum(jnp.where(row_i == col_i, pages[u], 0.0),
                                axis=0)
                f_rows = [f_pos]
                for n in range(_NEG):
                    m = u * _NEG + n
                    blk = pages[int(bidx[k, m])]
                    sq = seq_ref[k, 0, m, :]
                    f_rows.append(
                        jnp.sum(jnp.where(row_i == sq[None, :], blk, 0.0),
                                axis=0))
                f = jnp.stack(f_rows, axis=0)
                mx = jnp.max(f, axis=0)
                lse = mx + jnp.log(jnp.sum(jnp.exp(f - mx), axis=0))
                lossp_ref[k, 0, u, :] = lse - f[0]
                acc = (f[0] >= jnp.max(f[1:], axis=0)).astype(jnp.float32)
                accp_ref[k, 0, u, :] = acc

    return _cpc_body


def kernel(z, c, W, b, nframes):
    del nframes
    length = z.shape[1] - _K
    bidx_np, seq_np = _indices(length)
    seq = jnp.asarray(seq_np)

    cc = c[:, :-_K, :].reshape(_S, _U * length, _CD)
    z4 = z.reshape(_S, _U, z.shape[1], _ZD)
    zsh = jnp.stack(
        [z4[:, :, k:length + k, :].reshape(_S, _U * length, _ZD)
         for k in range(1, _K + 1)], axis=0)
    b2 = b.reshape(_K, 1, _ZD)

    grid = (_S,)
    lossp, accp = pl.pallas_call(
        _make_body(bidx_np),
        grid=grid,
        in_specs=[
            pl.BlockSpec((1, _U * length, _CD), lambda s: (s, 0, 0)),
            pl.BlockSpec((_K, _CD, _ZD), lambda s: (0, 0, 0)),
            pl.BlockSpec((_K, 1, _ZD), lambda s: (0, 0, 0)),
            pl.BlockSpec((_K, 1, _U * length, _ZD), lambda s: (0, s, 0, 0)),
            pl.BlockSpec((_K, 1, _U * _NEG, length), lambda s: (0, s, 0, 0)),
        ],
        out_specs=[
            pl.BlockSpec((_K, 1, _U, length), lambda s: (0, s, 0, 0)),
            pl.BlockSpec((_K, 1, _U, length), lambda s: (0, s, 0, 0)),
        ],
        out_shape=[
            jax.ShapeDtypeStruct((_K, _S, _U, length), jnp.float32),
            jax.ShapeDtypeStruct((_K, _S, _U, length), jnp.float32),
        ],
        scratch_shapes=[pltpu.VMEM((_U, length, _U * length), jnp.float32),
                        pltpu.VMEM((_U, length, _U * length), jnp.float32)],
    )(cc, W, b2, zsh, seq)

    denom = float(_S * _U * length)
    losses = lossp.sum(axis=(1, 2, 3)) / denom
    accs = accp.sum(axis=(1, 2, 3)) / denom
    return losses.mean(), accs
